# Initial kernel scaffold; baseline (speedup 1.0000x reference)
import functools

import jax
import jax.numpy as jnp
from jax import lax
from jax.experimental import pallas as pl
from jax.experimental.pallas import tpu as pltpu
from jax.experimental.pallas import tpu_sc as plsc

N_NODES = 10000
N_EDGES = 320000
N_FEAT = 128
N_HIDDEN = 128
LRELU_ALPHA = 0.05

NC = 2
NS = 16
NW = NC * NS
CE = N_EDGES // NW
K = 80
CHUNKS = CE // K
ROWS_PER_TILE = N_NODES // NS


def _tc1_body(x_ref, wt_ref, ap_ref, bv_ref, z_ref, s_ref):
    z = jnp.dot(x_ref[...], wt_ref[...], preferred_element_type=jnp.float32)
    z_ref[...] = z
    s_ref[...] = (
        jnp.dot(z, ap_ref[...], preferred_element_type=jnp.float32)
        + bv_ref[...]
    )


def _tc1(x, Wt, Apad, bvec):
    blk = 1000
    grid = (N_NODES // blk,)
    return pl.pallas_call(
        _tc1_body,
        grid=grid,
        in_specs=[
            pl.BlockSpec((blk, N_FEAT), lambda i: (i, 0)),
            pl.BlockSpec((N_FEAT, N_HIDDEN), lambda i: (0, 0)),
            pl.BlockSpec((N_HIDDEN, N_HIDDEN), lambda i: (0, 0)),
            pl.BlockSpec((1, N_HIDDEN), lambda i: (0, 0)),
        ],
        out_specs=[
            pl.BlockSpec((blk, N_HIDDEN), lambda i: (i, 0)),
            pl.BlockSpec((blk, N_HIDDEN), lambda i: (i, 0)),
        ],
        out_shape=[
            jax.ShapeDtypeStruct((N_NODES, N_HIDDEN), jnp.float32),
            jax.ShapeDtypeStruct((N_NODES, N_HIDDEN), jnp.float32),
        ],
    )(x, Wt, Apad, bvec)


def _tc2_body(a_ref, b_ref, o_ref):
    o_ref[...] = a_ref[...] + b_ref[...]


def _tc2(p0, p1):
    blk = 1000
    grid = (N_NODES // blk,)
    return pl.pallas_call(
        _tc2_body,
        grid=grid,
        in_specs=[
            pl.BlockSpec((blk, N_HIDDEN), lambda i: (i, 0)),
            pl.BlockSpec((blk, N_HIDDEN), lambda i: (i, 0)),
        ],
        out_specs=pl.BlockSpec((blk, N_HIDDEN), lambda i: (i, 0)),
        out_shape=jax.ShapeDtypeStruct((N_NODES, N_HIDDEN), jnp.float32),
    )(p0, p1)


_SC_MESH = plsc.VectorSubcoreMesh(core_axis_name="c", subcore_axis_name="s")


def _sc1_body(s1_hbm, s2_hbm, src_hbm, dst_hbm, zeros1_hbm,
              h_hbm, hsum_parts_hbm,
              s1_v, s2_v, srcc_v, dstc_v, hc_v, hsum_sh):
    cid = lax.axis_index("c")
    sid = lax.axis_index("s")
    wid = cid * NS + sid
    base = wid * CE

    @pl.when(sid == 0)
    def _():
        pltpu.sync_copy(zeros1_hbm, hsum_sh)

    pltpu.sync_copy(s1_hbm, s1_v)
    pltpu.sync_copy(s2_hbm, s2_v)
    plsc.subcore_barrier()

    def chunk(c, carry):
        off = base + c * K
        pltpu.sync_copy(src_hbm.at[pl.ds(off, K)], srcc_v)
        pltpu.sync_copy(dst_hbm.at[pl.ds(off, K)], dstc_v)
        for j in range(K // 16):
            sl = pl.ds(j * 16, 16)
            sidx = srcc_v[sl]
            didx = dstc_v[sl]
            t = plsc.load_gather(s1_v, [sidx]) + plsc.load_gather(s2_v, [didx])
            t = jnp.where(t > 0, t, LRELU_ALPHA * t)
            hc_v[sl] = jnp.exp(t)
        pltpu.sync_copy(hc_v, h_hbm.at[pl.ds(off, K)])
        pltpu.sync_copy(hc_v, hsum_sh.at[srcc_v], add=True)
        return carry

    lax.fori_loop(0, CHUNKS, chunk, 0)
    plsc.subcore_barrier()

    @pl.when(sid == 0)
    def _():
        pltpu.sync_copy(hsum_sh, hsum_parts_hbm.at[cid])


def _sc1(s1, s2, src, dst, zeros1):
    return pl.kernel(
        _sc1_body,
        out_type=[
            jax.ShapeDtypeStruct((N_EDGES,), jnp.float32),
            jax.ShapeDtypeStruct((NC, N_NODES), jnp.float32),
        ],
        mesh=_SC_MESH,
        scratch_types=[
            pltpu.VMEM((N_NODES,), jnp.float32),
            pltpu.VMEM((N_NODES,), jnp.float32),
            pltpu.VMEM((K,), jnp.int32),
            pltpu.VMEM((K,), jnp.int32),
            pltpu.VMEM((K,), jnp.float32),
            pltpu.VMEM_SHARED((N_NODES,), jnp.float32),
        ],
    )(s1, s2, src, dst, zeros1)


def _sc2_body(h_hbm, hsum_parts_hbm, src_hbm, dst_hbm, z_hbm, zeros2_hbm,
              alpha_hbm, out_parts_hbm,
              hsum_v, tmp_v, srcc_v, dstc_v, hc_v, rows_v, sem, out_sh):
    cid = lax.axis_index("c")
    sid = lax.axis_index("s")
    wid = cid * NS + sid
    base = wid * CE

    rsl = pl.ds(sid * ROWS_PER_TILE, ROWS_PER_TILE)
    pltpu.sync_copy(zeros2_hbm.at[rsl], out_sh.at[rsl])

    pltpu.sync_copy(hsum_parts_hbm.at[0], hsum_v)
    pltpu.sync_copy(hsum_parts_hbm.at[1], tmp_v)

    def addv(i, carry):
        sl = pl.ds(i * 16, 16)
        hsum_v[sl] = hsum_v[sl] + tmp_v[sl]
        return carry

    lax.fori_loop(0, N_NODES // 16, addv, 0)
    plsc.subcore_barrier()

    def chunk(c, carry):
        off = base + c * K
        pltpu.sync_copy(src_hbm.at[pl.ds(off, K)], srcc_v)
        pltpu.sync_copy(dst_hbm.at[pl.ds(off, K)], dstc_v)
        pltpu.sync_copy(h_hbm.at[pl.ds(off, K)], hc_v)
        pltpu.async_copy(z_hbm.at[dstc_v], rows_v, sem).wait()
        for j in range(K // 16):
            sl = pl.ds(j * 16, 16)
            hs = plsc.load_gather(hsum_v, [srcc_v[sl]])
            hc_v[sl] = hc_v[sl] / hs
        pltpu.sync_copy(hc_v, alpha_hbm.at[pl.ds(off, K)])

        def scale(r, carry2):
            a = hc_v[r]
            for jj in range(N_HIDDEN // 16):
                csl = pl.ds(jj * 16, 16)
                rows_v[r, csl] = rows_v[r, csl] * a
            return carry2

        lax.fori_loop(0, K, scale, 0)
        pltpu.sync_copy(rows_v, out_sh.at[srcc_v], add=True)
        return carry

    lax.fori_loop(0, CHUNKS, chunk, 0)
    plsc.subcore_barrier()
    pltpu.sync_copy(out_sh.at[rsl], out_parts_hbm.at[cid, rsl])


def _sc2(h, hsum_parts, src, dst, z, zeros2):
    return pl.kernel(
        _sc2_body,
        out_type=[
            jax.ShapeDtypeStruct((N_EDGES,), jnp.float32),
            jax.ShapeDtypeStruct((NC, N_NODES, N_HIDDEN), jnp.float32),
        ],
        mesh=_SC_MESH,
        scratch_types=[
            pltpu.VMEM((N_NODES,), jnp.float32),
            pltpu.VMEM((N_NODES,), jnp.float32),
            pltpu.VMEM((K,), jnp.int32),
            pltpu.VMEM((K,), jnp.int32),
            pltpu.VMEM((K,), jnp.float32),
            pltpu.VMEM((K, N_HIDDEN), jnp.float32),
            pltpu.SemaphoreType.DMA,
            pltpu.VMEM_SHARED((N_NODES, N_HIDDEN), jnp.float32),
        ],
    )(h, hsum_parts, src, dst, z, zeros2)


@jax.jit
def _run(x, edge_index, W, a_w, a_b):
    src = edge_index[0, :].astype(jnp.int32)
    dst = edge_index[1, :].astype(jnp.int32)

    a1 = a_w[0, :N_HIDDEN]
    a2 = a_w[0, N_HIDDEN:]
    Apad = jnp.zeros((N_HIDDEN, N_HIDDEN), jnp.float32)
    Apad = Apad.at[:, 0].set(a1).at[:, 1].set(a2)
    bvec = jnp.zeros((1, N_HIDDEN), jnp.float32).at[0, 0].set(a_b[0])

    z, S = _tc1(x, W.T, Apad, bvec)
    s1 = S[:, 0]
    s2 = S[:, 1]

    zeros1 = jnp.zeros((N_NODES,), jnp.float32)
    h, hsum_parts = _sc1(s1, s2, src, dst, zeros1)

    zeros2 = jnp.zeros((N_NODES, N_HIDDEN), jnp.float32)
    alpha, out_parts = _sc2(h, hsum_parts, src, dst, z, zeros2)

    out = _tc2(out_parts[0], out_parts[1])
    return out, alpha


def kernel(x, edge_index, W, a_w, a_b):
    return _run(x, edge_index, W, a_w, a_b)


# speedup vs baseline: 7.0293x; 7.0293x over previous
import functools

import jax
import jax.numpy as jnp
from jax import lax
from jax.experimental import pallas as pl
from jax.experimental.pallas import tpu as pltpu
from jax.experimental.pallas import tpu_sc as plsc

N_NODES = 10000
N_EDGES = 320000
N_FEAT = 128
N_HIDDEN = 128
LRELU_ALPHA = 0.05

NC = 2
NS = 16
NW = NC * NS
CE = N_EDGES // NW
K = 80
CHUNKS = CE // K
ROWS_PER_TILE = 624
TAIL_ROWS = N_NODES - NS * ROWS_PER_TILE


def _tc1_body(x_ref, wt_ref, ap_ref, bv_ref, z_ref, s_ref):
    z = jnp.dot(x_ref[...], wt_ref[...], preferred_element_type=jnp.float32)
    z_ref[...] = z
    s_ref[...] = (
        jnp.dot(z, ap_ref[...], preferred_element_type=jnp.float32)
        + bv_ref[...]
    )


def _tc1(x, Wt, Apad, bvec):
    blk = 1000
    grid = (N_NODES // blk,)
    return pl.pallas_call(
        _tc1_body,
        grid=grid,
        in_specs=[
            pl.BlockSpec((blk, N_FEAT), lambda i: (i, 0)),
            pl.BlockSpec((N_FEAT, N_HIDDEN), lambda i: (0, 0)),
            pl.BlockSpec((N_HIDDEN, N_HIDDEN), lambda i: (0, 0)),
            pl.BlockSpec((1, N_HIDDEN), lambda i: (0, 0)),
        ],
        out_specs=[
            pl.BlockSpec((blk, N_HIDDEN), lambda i: (i, 0)),
            pl.BlockSpec((blk, N_HIDDEN), lambda i: (i, 0)),
        ],
        out_shape=[
            jax.ShapeDtypeStruct((N_NODES, N_HIDDEN), jnp.float32),
            jax.ShapeDtypeStruct((N_NODES, N_HIDDEN), jnp.float32),
        ],
    )(x, Wt, Apad, bvec)


def _tc2_body(a_ref, b_ref, o_ref):
    o_ref[...] = a_ref[...] + b_ref[...]


def _tc2(p0, p1):
    blk = 1000
    grid = (N_NODES // blk,)
    return pl.pallas_call(
        _tc2_body,
        grid=grid,
        in_specs=[
            pl.BlockSpec((blk, N_HIDDEN), lambda i: (i, 0)),
            pl.BlockSpec((blk, N_HIDDEN), lambda i: (i, 0)),
        ],
        out_specs=pl.BlockSpec((blk, N_HIDDEN), lambda i: (i, 0)),
        out_shape=jax.ShapeDtypeStruct((N_NODES, N_HIDDEN), jnp.float32),
    )(p0, p1)


_SC_MESH = plsc.VectorSubcoreMesh(core_axis_name="c", subcore_axis_name="s")


def _sc1_body(s1_hbm, s2_hbm, src_hbm, dst_hbm, zeros1_hbm,
              h_hbm, hsum_parts_hbm,
              s1_v, s2_v, srcc_v, dstc_v, hc_v, hsum_sh):
    cid = lax.axis_index("c")
    sid = lax.axis_index("s")
    wid = cid * NS + sid
    base = wid * CE

    @pl.when(sid == 0)
    def _():
        pltpu.sync_copy(zeros1_hbm, hsum_sh)

    pltpu.sync_copy(s1_hbm, s1_v)
    pltpu.sync_copy(s2_hbm, s2_v)
    plsc.subcore_barrier()

    def chunk(c, carry):
        off = base + c * K
        pltpu.sync_copy(src_hbm.at[pl.ds(off, K)], srcc_v)
        pltpu.sync_copy(dst_hbm.at[pl.ds(off, K)], dstc_v)
        for j in range(K // 16):
            sl = pl.ds(j * 16, 16)
            sidx = srcc_v[sl]
            didx = dstc_v[sl]
            t = plsc.load_gather(s1_v, [sidx]) + plsc.load_gather(s2_v, [didx])
            t = jnp.where(t > 0, t, LRELU_ALPHA * t)
            hc_v[sl] = jnp.exp(t)
        pltpu.sync_copy(hc_v, h_hbm.at[pl.ds(off, K)])
        pltpu.sync_copy(hc_v, hsum_sh.at[srcc_v], add=True)
        return carry

    lax.fori_loop(0, CHUNKS, chunk, 0)
    plsc.subcore_barrier()

    @pl.when(sid == 0)
    def _():
        pltpu.sync_copy(hsum_sh, hsum_parts_hbm.at[cid])


def _sc1(s1, s2, src, dst, zeros1):
    return pl.kernel(
        _sc1_body,
        out_type=[
            jax.ShapeDtypeStruct((N_EDGES,), jnp.float32),
            jax.ShapeDtypeStruct((NC, N_NODES), jnp.float32),
        ],
        mesh=_SC_MESH,
        compiler_params=pltpu.CompilerParams(needs_layout_passes=False),
        scratch_types=[
            pltpu.VMEM((N_NODES,), jnp.float32),
            pltpu.VMEM((N_NODES,), jnp.float32),
            pltpu.VMEM((K,), jnp.int32),
            pltpu.VMEM((K,), jnp.int32),
            pltpu.VMEM((K,), jnp.float32),
            pltpu.VMEM_SHARED((N_NODES,), jnp.float32),
        ],
    )(s1, s2, src, dst, zeros1)


def _sc2_body(h_hbm, hsum_parts_hbm, src_hbm, dst_hbm, z_hbm, zeros2_hbm,
              alpha_hbm, out_parts_hbm,
              hsum_v, tmp_v, srcc_v, dstc_v, hc_v, rows_v, sem, out_sh):
    cid = lax.axis_index("c")
    sid = lax.axis_index("s")
    wid = cid * NS + sid
    base = wid * CE

    rsl = pl.ds(sid * ROWS_PER_TILE, ROWS_PER_TILE)
    tsl = pl.ds(NS * ROWS_PER_TILE, TAIL_ROWS)
    pltpu.sync_copy(zeros2_hbm.at[rsl], out_sh.at[rsl])

    @pl.when(sid == 0)
    def _():
        pltpu.sync_copy(zeros2_hbm.at[tsl], out_sh.at[tsl])

    pltpu.sync_copy(hsum_parts_hbm.at[0], hsum_v)
    pltpu.sync_copy(hsum_parts_hbm.at[1], tmp_v)

    def addv(i, carry):
        sl = pl.ds(i * 16, 16)
        hsum_v[sl] = hsum_v[sl] + tmp_v[sl]
        return carry

    lax.fori_loop(0, N_NODES // 16, addv, 0)
    plsc.subcore_barrier()

    def chunk(c, carry):
        off = base + c * K
        pltpu.sync_copy(src_hbm.at[pl.ds(off, K)], srcc_v)
        pltpu.sync_copy(dst_hbm.at[pl.ds(off, K)], dstc_v)
        pltpu.sync_copy(h_hbm.at[pl.ds(off, K)], hc_v)
        pltpu.async_copy(z_hbm.at[dstc_v], rows_v, sem).wait()
        for j in range(K // 16):
            sl = pl.ds(j * 16, 16)
            hs = plsc.load_gather(hsum_v, [srcc_v[sl]])
            hc_v[sl] = hc_v[sl] / hs
        pltpu.sync_copy(hc_v, alpha_hbm.at[pl.ds(off, K)])

        def scale(r, carry2):
            a = plsc.load_gather(hc_v, [jnp.full((16,), r, jnp.int32)])
            for jj in range(N_HIDDEN // 16):
                csl = pl.ds(jj * 16, 16)
                rows_v[r, csl] = rows_v[r, csl] * a
            return carry2

        lax.fori_loop(0, K, scale, 0)
        pltpu.sync_copy(rows_v, out_sh.at[srcc_v], add=True)
        return carry

    lax.fori_loop(0, CHUNKS, chunk, 0)
    plsc.subcore_barrier()
    pltpu.sync_copy(out_sh.at[rsl], out_parts_hbm.at[cid, rsl])

    @pl.when(sid == 0)
    def _():
        pltpu.sync_copy(out_sh.at[tsl], out_parts_hbm.at[cid, tsl])


def _sc2(h, hsum_parts, src, dst, z, zeros2):
    return pl.kernel(
        _sc2_body,
        out_type=[
            jax.ShapeDtypeStruct((N_EDGES,), jnp.float32),
            jax.ShapeDtypeStruct((NC, N_NODES, N_HIDDEN), jnp.float32),
        ],
        mesh=_SC_MESH,
        compiler_params=pltpu.CompilerParams(needs_layout_passes=False),
        scratch_types=[
            pltpu.VMEM((N_NODES,), jnp.float32),
            pltpu.VMEM((N_NODES,), jnp.float32),
            pltpu.VMEM((K,), jnp.int32),
            pltpu.VMEM((K,), jnp.int32),
            pltpu.VMEM((K,), jnp.float32),
            pltpu.VMEM((K, N_HIDDEN), jnp.float32),
            pltpu.SemaphoreType.DMA,
            pltpu.VMEM_SHARED((N_NODES, N_HIDDEN), jnp.float32),
        ],
    )(h, hsum_parts, src, dst, z, zeros2)


@jax.jit
def _run(x, edge_index, W, a_w, a_b):
    src = edge_index[0, :].astype(jnp.int32)
    dst = edge_index[1, :].astype(jnp.int32)

    a1 = a_w[0, :N_HIDDEN]
    a2 = a_w[0, N_HIDDEN:]
    Apad = jnp.zeros((N_HIDDEN, N_HIDDEN), jnp.float32)
    Apad = Apad.at[:, 0].set(a1).at[:, 1].set(a2)
    bvec = jnp.zeros((1, N_HIDDEN), jnp.float32).at[0, 0].set(a_b[0])

    z, S = _tc1(x, W.T, Apad, bvec)
    s1 = S[:, 0]
    s2 = S[:, 1]

    zeros1 = jnp.zeros((N_NODES,), jnp.float32)
    h, hsum_parts = _sc1(s1, s2, src, dst, zeros1)

    zeros2 = jnp.zeros((N_NODES, N_HIDDEN), jnp.float32)
    alpha, out_parts = _sc2(h, hsum_parts, src, dst, z, zeros2)

    out = _tc2(out_parts[0], out_parts[1])
    return out, alpha


def kernel(x, edge_index, W, a_w, a_b):
    return _run(x, edge_index, W, a_w, a_b)


# speedup vs baseline: 14.5484x; 2.0697x over previous
import functools

import jax
import jax.numpy as jnp
from jax import lax
from jax.experimental import pallas as pl
from jax.experimental.pallas import tpu as pltpu
from jax.experimental.pallas import tpu_sc as plsc

N_NODES = 10000
N_EDGES = 320000
N_FEAT = 128
N_HIDDEN = 128
LRELU_ALPHA = 0.05

NC = 2
NS = 16
NW = NC * NS
CE = N_EDGES // NW
K = 80
CHUNKS = CE // K
PAIRS = (CHUNKS - 1) // 2
ROWS_PER_TILE = 624
TAIL_ROWS = N_NODES - NS * ROWS_PER_TILE


def _tc1_body(x_ref, wt_ref, ap_ref, bv_ref, z_ref, s_ref):
    z = jnp.dot(x_ref[...], wt_ref[...], preferred_element_type=jnp.float32)
    z_ref[...] = z
    s_ref[...] = (
        jnp.dot(z, ap_ref[...], preferred_element_type=jnp.float32)
        + bv_ref[...]
    )


def _tc1(x, Wt, Apad, bvec):
    blk = 1000
    grid = (N_NODES // blk,)
    return pl.pallas_call(
        _tc1_body,
        grid=grid,
        in_specs=[
            pl.BlockSpec((blk, N_FEAT), lambda i: (i, 0)),
            pl.BlockSpec((N_FEAT, N_HIDDEN), lambda i: (0, 0)),
            pl.BlockSpec((N_HIDDEN, N_HIDDEN), lambda i: (0, 0)),
            pl.BlockSpec((1, N_HIDDEN), lambda i: (0, 0)),
        ],
        out_specs=[
            pl.BlockSpec((blk, N_HIDDEN), lambda i: (i, 0)),
            pl.BlockSpec((blk, N_HIDDEN), lambda i: (i, 0)),
        ],
        out_shape=[
            jax.ShapeDtypeStruct((N_NODES, N_HIDDEN), jnp.float32),
            jax.ShapeDtypeStruct((N_NODES, N_HIDDEN), jnp.float32),
        ],
    )(x, Wt, Apad, bvec)


def _tc2_body(a_ref, b_ref, o_ref):
    o_ref[...] = a_ref[...] + b_ref[...]


def _tc2(p0, p1):
    blk = 1000
    grid = (N_NODES // blk,)
    return pl.pallas_call(
        _tc2_body,
        grid=grid,
        in_specs=[
            pl.BlockSpec((blk, N_HIDDEN), lambda i: (i, 0)),
            pl.BlockSpec((blk, N_HIDDEN), lambda i: (i, 0)),
        ],
        out_specs=pl.BlockSpec((blk, N_HIDDEN), lambda i: (i, 0)),
        out_shape=jax.ShapeDtypeStruct((N_NODES, N_HIDDEN), jnp.float32),
    )(p0, p1)


_SC_MESH = plsc.VectorSubcoreMesh(core_axis_name="c", subcore_axis_name="s")


def _sc1_body(s1_hbm, s2_hbm, src_hbm, dst_hbm, zeros1_hbm,
              h_hbm, hsum_parts_hbm,
              s1_v, s2_v, srcall_v, dstall_v, hall_v, srcc2_v,
              sem_a0, sem_a1, hsum_sh):
    cid = lax.axis_index("c")
    sid = lax.axis_index("s")
    wid = cid * NS + sid
    base = wid * CE

    @pl.when(sid == 0)
    def _():
        pltpu.sync_copy(zeros1_hbm, hsum_sh)

    pltpu.sync_copy(s1_hbm, s1_v)
    pltpu.sync_copy(s2_hbm, s2_v)
    pltpu.sync_copy(src_hbm.at[pl.ds(base, CE)], srcall_v)
    pltpu.sync_copy(dst_hbm.at[pl.ds(base, CE)], dstall_v)
    plsc.subcore_barrier()

    sems = (sem_a0, sem_a1)

    def compute_chunk(c):
        for j in range(K // 16):
            sl = pl.ds(c * K + j * 16, 16)
            t = (plsc.load_gather(s1_v, [srcall_v[sl]])
                 + plsc.load_gather(s2_v, [dstall_v[sl]]))
            t = jnp.where(t > 0, t, LRELU_ALPHA * t)
            hall_v[sl] = jnp.exp(t)

    def issue_scatter(c, b):
        for j in range(K // 16):
            srcc2_v[b, pl.ds(j * 16, 16)] = srcall_v[pl.ds(c * K + j * 16, 16)]
        pltpu.async_copy(
            hall_v.at[pl.ds(c * K, K)],
            hsum_sh.at[srcc2_v.at[b]],
            sems[b],
            add=True,
        )

    def wait_scatter(c, b):
        pltpu.make_async_copy(
            hall_v.at[pl.ds(c * K, K)],
            hsum_sh.at[srcc2_v.at[b]],
            sems[b],
        ).wait()

    def pair(c0, carry):
        for b in range(2):
            c = 2 * c0 + b
            compute_chunk(c)

            @pl.when(c >= 2)
            def _():
                wait_scatter(c - 2, b)

            issue_scatter(c, b)
        return carry

    lax.fori_loop(0, PAIRS, pair, 0)
    compute_chunk(CHUNKS - 1)
    wait_scatter(CHUNKS - 3, 0)
    issue_scatter(CHUNKS - 1, 0)
    wait_scatter(CHUNKS - 2, 1)
    wait_scatter(CHUNKS - 1, 0)

    pltpu.sync_copy(hall_v, h_hbm.at[pl.ds(base, CE)])
    plsc.subcore_barrier()

    @pl.when(sid == 0)
    def _():
        pltpu.sync_copy(hsum_sh, hsum_parts_hbm.at[cid])


def _sc1(s1, s2, src, dst, zeros1):
    return pl.kernel(
        _sc1_body,
        out_type=[
            jax.ShapeDtypeStruct((N_EDGES,), jnp.float32),
            jax.ShapeDtypeStruct((NC, N_NODES), jnp.float32),
        ],
        mesh=_SC_MESH,
        compiler_params=pltpu.CompilerParams(needs_layout_passes=False),
        scratch_types=[
            pltpu.VMEM((N_NODES,), jnp.float32),
            pltpu.VMEM((N_NODES,), jnp.float32),
            pltpu.VMEM((CE,), jnp.int32),
            pltpu.VMEM((CE,), jnp.int32),
            pltpu.VMEM((CE,), jnp.float32),
            pltpu.VMEM((2, K), jnp.int32),
            pltpu.SemaphoreType.DMA,
            pltpu.SemaphoreType.DMA,
            pltpu.VMEM_SHARED((N_NODES,), jnp.float32),
        ],
    )(s1, s2, src, dst, zeros1)


def _sc2_body(h_hbm, hsum_parts_hbm, src_hbm, dst_hbm, z_hbm, zeros2_hbm,
              alpha_hbm, out_parts_hbm,
              hsum_v, tmp_v, srcc2_v, dstc2_v, hc2_v, rows0_v, rows1_v,
              sem_i0, sem_i1, sem_g0, sem_g1, sem_s0, sem_s1,
              sem_a0, sem_a1, out_sh):
    cid = lax.axis_index("c")
    sid = lax.axis_index("s")
    wid = cid * NS + sid
    base = wid * CE

    rsl = pl.ds(sid * ROWS_PER_TILE, ROWS_PER_TILE)
    tsl = pl.ds(NS * ROWS_PER_TILE, TAIL_ROWS)
    pltpu.sync_copy(zeros2_hbm.at[rsl], out_sh.at[rsl])

    @pl.when(sid == 0)
    def _():
        pltpu.sync_copy(zeros2_hbm.at[tsl], out_sh.at[tsl])

    pltpu.sync_copy(hsum_parts_hbm.at[0], hsum_v)
    pltpu.sync_copy(hsum_parts_hbm.at[1], tmp_v)

    def addv(i, carry):
        sl = pl.ds(i * 16, 16)
        hsum_v[sl] = hsum_v[sl] + tmp_v[sl]
        return carry

    lax.fori_loop(0, N_NODES // 16, addv, 0)
    plsc.subcore_barrier()

    rows = (rows0_v, rows1_v)
    sem_i = (sem_i0, sem_i1)
    sem_g = (sem_g0, sem_g1)
    sem_s = (sem_s0, sem_s1)
    sem_a = (sem_a0, sem_a1)

    def issue_stage(c, b):
        off = base + c * K
        pltpu.async_copy(src_hbm.at[pl.ds(off, K)], srcc2_v.at[b], sem_i[b])
        pltpu.async_copy(dst_hbm.at[pl.ds(off, K)], dstc2_v.at[b], sem_i[b])
        pltpu.async_copy(h_hbm.at[pl.ds(off, K)], hc2_v.at[b], sem_i[b])

    def wait_stage(c, b):
        off = base + c * K
        pltpu.make_async_copy(
            src_hbm.at[pl.ds(off, K)], srcc2_v.at[b], sem_i[b]).wait()
        pltpu.make_async_copy(
            dst_hbm.at[pl.ds(off, K)], dstc2_v.at[b], sem_i[b]).wait()
        pltpu.make_async_copy(
            h_hbm.at[pl.ds(off, K)], hc2_v.at[b], sem_i[b]).wait()

    def issue_gather(c, b):
        pltpu.async_copy(z_hbm.at[dstc2_v.at[b]], rows[b], sem_g[b])

    def wait_gather(c, b):
        pltpu.make_async_copy(
            z_hbm.at[dstc2_v.at[b]], rows[b], sem_g[b]).wait()

    def issue_scatter(c, b):
        pltpu.async_copy(rows[b], out_sh.at[srcc2_v.at[b]], sem_s[b], add=True)

    def wait_scatter(c, b):
        pltpu.make_async_copy(
            rows[b], out_sh.at[srcc2_v.at[b]], sem_s[b]).wait()

    def issue_alpha(c, b):
        pltpu.async_copy(
            hc2_v.at[b], alpha_hbm.at[pl.ds(base + c * K, K)], sem_a[b])

    def wait_alpha(c, b):
        pltpu.make_async_copy(
            hc2_v.at[b], alpha_hbm.at[pl.ds(base + c * K, K)], sem_a[b]).wait()

    def compute_alpha(b):
        for j in range(K // 16):
            sl = pl.ds(j * 16, 16)
            hs = plsc.load_gather(hsum_v, [srcc2_v[b, sl]])
            hc2_v[b, sl] = hc2_v[b, sl] / hs

    def scale_rows(b):
        rows_v = rows[b]
        hrow = hc2_v.at[b]

        def scale(r, carry2):
            a = plsc.load_gather(hrow, [jnp.full((16,), 0, jnp.int32) + r])
            for jj in range(N_HIDDEN // 16):
                csl = pl.ds(jj * 16, 16)
                rows_v[r, csl] = rows_v[r, csl] * a
            return carry2

        lax.fori_loop(0, K, scale, 0)

    def step(c, b):
        @pl.when(c >= 1)
        def _():
            wait_scatter(c - 1, 1 - b)
            wait_alpha(c - 1, 1 - b)

        @pl.when(c + 1 < CHUNKS)
        def _():
            issue_stage(c + 1, 1 - b)

        wait_gather(c, b)
        compute_alpha(b)
        issue_alpha(c, b)

        @pl.when(c + 1 < CHUNKS)
        def _():
            wait_stage(c + 1, 1 - b)
            issue_gather(c + 1, 1 - b)

        scale_rows(b)
        issue_scatter(c, b)

    issue_stage(0, 0)
    wait_stage(0, 0)
    issue_gather(0, 0)

    def pair(c0, carry):
        for b in range(2):
            step(2 * c0 + b, b)
        return carry

    lax.fori_loop(0, PAIRS, pair, 0)
    step(CHUNKS - 1, 0)
    wait_scatter(CHUNKS - 1, 0)
    wait_alpha(CHUNKS - 1, 0)

    plsc.subcore_barrier()
    pltpu.sync_copy(out_sh.at[rsl], out_parts_hbm.at[cid, rsl])

    @pl.when(sid == 0)
    def _():
        pltpu.sync_copy(out_sh.at[tsl], out_parts_hbm.at[cid, tsl])


def _sc2(h, hsum_parts, src, dst, z, zeros2):
    return pl.kernel(
        _sc2_body,
        out_type=[
            jax.ShapeDtypeStruct((N_EDGES,), jnp.float32),
            jax.ShapeDtypeStruct((NC, N_NODES, N_HIDDEN), jnp.float32),
        ],
        mesh=_SC_MESH,
        compiler_params=pltpu.CompilerParams(needs_layout_passes=False),
        scratch_types=[
            pltpu.VMEM((N_NODES,), jnp.float32),
            pltpu.VMEM((N_NODES,), jnp.float32),
            pltpu.VMEM((2, K), jnp.int32),
            pltpu.VMEM((2, K), jnp.int32),
            pltpu.VMEM((2, K), jnp.float32),
            pltpu.VMEM((K, N_HIDDEN), jnp.float32),
            pltpu.VMEM((K, N_HIDDEN), jnp.float32),
            pltpu.SemaphoreType.DMA,
            pltpu.SemaphoreType.DMA,
            pltpu.SemaphoreType.DMA,
            pltpu.SemaphoreType.DMA,
            pltpu.SemaphoreType.DMA,
            pltpu.SemaphoreType.DMA,
            pltpu.SemaphoreType.DMA,
            pltpu.SemaphoreType.DMA,
            pltpu.VMEM_SHARED((N_NODES, N_HIDDEN), jnp.float32),
        ],
    )(h, hsum_parts, src, dst, z, zeros2)


@jax.jit
def _run(x, edge_index, W, a_w, a_b):
    src = edge_index[0, :].astype(jnp.int32)
    dst = edge_index[1, :].astype(jnp.int32)

    a1 = a_w[0, :N_HIDDEN]
    a2 = a_w[0, N_HIDDEN:]
    Apad = jnp.zeros((N_HIDDEN, N_HIDDEN), jnp.float32)
    Apad = Apad.at[:, 0].set(a1).at[:, 1].set(a2)
    bvec = jnp.zeros((1, N_HIDDEN), jnp.float32).at[0, 0].set(a_b[0])

    z, S = _tc1(x, W.T, Apad, bvec)
    s1 = S[:, 0]
    s2 = S[:, 1]

    zeros1 = jnp.zeros((N_NODES,), jnp.float32)
    h, hsum_parts = _sc1(s1, s2, src, dst, zeros1)

    zeros2 = jnp.zeros((N_NODES, N_HIDDEN), jnp.float32)
    alpha, out_parts = _sc2(h, hsum_parts, src, dst, z, zeros2)

    out = _tc2(out_parts[0], out_parts[1])
    return out, alpha


def kernel(x, edge_index, W, a_w, a_b):
    return _run(x, edge_index, W, a_w, a_b)


# speedup vs baseline: 17.0301x; 1.1706x over previous
import functools

import jax
import jax.numpy as jnp
from jax import lax
from jax.experimental import pallas as pl
from jax.experimental.pallas import tpu as pltpu
from jax.experimental.pallas import tpu_sc as plsc

N_NODES = 10000
N_EDGES = 320000
N_FEAT = 128
N_HIDDEN = 128
LRELU_ALPHA = 0.05

NC = 2
NS = 16
NW = NC * NS
CE = N_EDGES // NW
K = 80
CHUNKS = CE // K
PAIRS = (CHUNKS - 1) // 2
ROWS_PER_TILE = 624
TAIL_ROWS = N_NODES - NS * ROWS_PER_TILE


def _tc1_body(x_ref, wt_ref, ap_ref, bv_ref, z_ref, s_ref):
    z = jnp.dot(x_ref[...], wt_ref[...], preferred_element_type=jnp.float32)
    z_ref[...] = z
    s_ref[...] = (
        jnp.dot(z, ap_ref[...], preferred_element_type=jnp.float32)
        + bv_ref[...]
    )


def _tc1(x, Wt, Apad, bvec):
    blk = 1000
    grid = (N_NODES // blk,)
    return pl.pallas_call(
        _tc1_body,
        grid=grid,
        in_specs=[
            pl.BlockSpec((blk, N_FEAT), lambda i: (i, 0)),
            pl.BlockSpec((N_FEAT, N_HIDDEN), lambda i: (0, 0)),
            pl.BlockSpec((N_HIDDEN, N_HIDDEN), lambda i: (0, 0)),
            pl.BlockSpec((1, N_HIDDEN), lambda i: (0, 0)),
        ],
        out_specs=[
            pl.BlockSpec((blk, N_HIDDEN), lambda i: (i, 0)),
            pl.BlockSpec((blk, N_HIDDEN), lambda i: (i, 0)),
        ],
        out_shape=[
            jax.ShapeDtypeStruct((N_NODES, N_HIDDEN), jnp.float32),
            jax.ShapeDtypeStruct((N_NODES, N_HIDDEN), jnp.float32),
        ],
    )(x, Wt, Apad, bvec)


def _tc2_body(a_ref, b_ref, o_ref):
    o_ref[...] = a_ref[...] + b_ref[...]


def _tc2(p0, p1):
    blk = 1000
    grid = (N_NODES // blk,)
    return pl.pallas_call(
        _tc2_body,
        grid=grid,
        in_specs=[
            pl.BlockSpec((blk, N_HIDDEN), lambda i: (i, 0)),
            pl.BlockSpec((blk, N_HIDDEN), lambda i: (i, 0)),
        ],
        out_specs=pl.BlockSpec((blk, N_HIDDEN), lambda i: (i, 0)),
        out_shape=jax.ShapeDtypeStruct((N_NODES, N_HIDDEN), jnp.float32),
    )(p0, p1)


_SC_MESH = plsc.VectorSubcoreMesh(core_axis_name="c", subcore_axis_name="s")


def _sc1_body(s1_hbm, s2_hbm, src_hbm, dst_hbm, zeros1_hbm,
              h_hbm, hsum_parts_hbm,
              s1_v, s2_v, srcall_v, dstall_v, hall_v, srcc2_v,
              sem_a0, sem_a1, hsum_sh):
    cid = lax.axis_index("c")
    sid = lax.axis_index("s")
    wid = cid * NS + sid
    base = wid * CE

    @pl.when(sid == 0)
    def _():
        pltpu.sync_copy(zeros1_hbm, hsum_sh)

    pltpu.sync_copy(s1_hbm, s1_v)
    pltpu.sync_copy(s2_hbm, s2_v)
    pltpu.sync_copy(src_hbm.at[pl.ds(base, CE)], srcall_v)
    pltpu.sync_copy(dst_hbm.at[pl.ds(base, CE)], dstall_v)
    plsc.subcore_barrier()

    sems = (sem_a0, sem_a1)

    def compute_chunk(c):
        for j in range(K // 16):
            sl = pl.ds(c * K + j * 16, 16)
            t = (plsc.load_gather(s1_v, [srcall_v[sl]])
                 + plsc.load_gather(s2_v, [dstall_v[sl]]))
            t = jnp.where(t > 0, t, LRELU_ALPHA * t)
            hall_v[sl] = jnp.exp(t)

    def issue_scatter(c, b):
        for j in range(K // 16):
            srcc2_v[b, pl.ds(j * 16, 16)] = srcall_v[pl.ds(c * K + j * 16, 16)]
        pltpu.async_copy(
            hall_v.at[pl.ds(c * K, K)],
            hsum_sh.at[srcc2_v.at[b]],
            sems[b],
            add=True,
        )

    def wait_scatter(c, b):
        pltpu.make_async_copy(
            hall_v.at[pl.ds(c * K, K)],
            hsum_sh.at[srcc2_v.at[b]],
            sems[b],
        ).wait()

    def pair(c0, carry):
        for b in range(2):
            c = 2 * c0 + b
            compute_chunk(c)

            @pl.when(c >= 2)
            def _():
                wait_scatter(c - 2, b)

            issue_scatter(c, b)
        return carry

    lax.fori_loop(0, PAIRS, pair, 0)
    compute_chunk(CHUNKS - 1)
    wait_scatter(CHUNKS - 3, 0)
    issue_scatter(CHUNKS - 1, 0)
    wait_scatter(CHUNKS - 2, 1)
    wait_scatter(CHUNKS - 1, 0)

    pltpu.sync_copy(hall_v, h_hbm.at[pl.ds(base, CE)])
    plsc.subcore_barrier()

    @pl.when(sid == 0)
    def _():
        pltpu.sync_copy(hsum_sh, hsum_parts_hbm.at[cid])


def _sc1(s1, s2, src, dst, zeros1):
    return pl.kernel(
        _sc1_body,
        out_type=[
            jax.ShapeDtypeStruct((N_EDGES,), jnp.float32),
            jax.ShapeDtypeStruct((NC, N_NODES), jnp.float32),
        ],
        mesh=_SC_MESH,
        compiler_params=pltpu.CompilerParams(needs_layout_passes=False),
        scratch_types=[
            pltpu.VMEM((N_NODES,), jnp.float32),
            pltpu.VMEM((N_NODES,), jnp.float32),
            pltpu.VMEM((CE,), jnp.int32),
            pltpu.VMEM((CE,), jnp.int32),
            pltpu.VMEM((CE,), jnp.float32),
            pltpu.VMEM((2, K), jnp.int32),
            pltpu.SemaphoreType.DMA,
            pltpu.SemaphoreType.DMA,
            pltpu.VMEM_SHARED((N_NODES,), jnp.float32),
        ],
    )(s1, s2, src, dst, zeros1)


def _sc2_body(h_hbm, hsum_parts_hbm, src_hbm, dst_hbm, z_hbm, zeros2_hbm,
              alpha_hbm, out_parts_hbm,
              hsum_v, tmp_v, srcc2_v, dstc2_v, hc2_v, rows0_v, rows1_v,
              sem_i0, sem_i1, sem_g0, sem_g1, sem_s0, sem_s1,
              sem_a0, sem_a1, out_sh):
    cid = lax.axis_index("c")
    sid = lax.axis_index("s")
    wid = cid * NS + sid
    base = wid * CE

    rsl = pl.ds(sid * ROWS_PER_TILE, ROWS_PER_TILE)
    tsl = pl.ds(NS * ROWS_PER_TILE, TAIL_ROWS)
    pltpu.sync_copy(zeros2_hbm.at[rsl], out_sh.at[rsl])

    @pl.when(sid == 0)
    def _():
        pltpu.sync_copy(zeros2_hbm.at[tsl], out_sh.at[tsl])

    pltpu.sync_copy(hsum_parts_hbm.at[0], hsum_v)
    pltpu.sync_copy(hsum_parts_hbm.at[1], tmp_v)

    def addv(i, carry):
        sl = pl.ds(i * 16, 16)
        hsum_v[sl] = hsum_v[sl] + tmp_v[sl]
        return carry

    lax.fori_loop(0, N_NODES // 16, addv, 0)
    plsc.subcore_barrier()

    rows = (rows0_v, rows1_v)
    sem_i = (sem_i0, sem_i1)
    sem_g = (sem_g0, sem_g1)
    sem_s = (sem_s0, sem_s1)
    sem_a = (sem_a0, sem_a1)

    def issue_stage(c, b):
        off = base + c * K
        pltpu.async_copy(src_hbm.at[pl.ds(off, K)], srcc2_v.at[b], sem_i[b])
        pltpu.async_copy(dst_hbm.at[pl.ds(off, K)], dstc2_v.at[b], sem_i[b])
        pltpu.async_copy(h_hbm.at[pl.ds(off, K)], hc2_v.at[b], sem_i[b])

    def wait_stage(c, b):
        off = base + c * K
        pltpu.make_async_copy(
            src_hbm.at[pl.ds(off, K)], srcc2_v.at[b], sem_i[b]).wait()
        pltpu.make_async_copy(
            dst_hbm.at[pl.ds(off, K)], dstc2_v.at[b], sem_i[b]).wait()
        pltpu.make_async_copy(
            h_hbm.at[pl.ds(off, K)], hc2_v.at[b], sem_i[b]).wait()

    def issue_gather(c, b):
        pltpu.async_copy(z_hbm.at[dstc2_v.at[b]], rows[b], sem_g[b])

    def wait_gather(c, b):
        pltpu.make_async_copy(
            z_hbm.at[dstc2_v.at[b]], rows[b], sem_g[b]).wait()

    def issue_scatter(c, b):
        pltpu.async_copy(rows[b], out_sh.at[srcc2_v.at[b]], sem_s[b], add=True)

    def wait_scatter(c, b):
        pltpu.make_async_copy(
            rows[b], out_sh.at[srcc2_v.at[b]], sem_s[b]).wait()

    def issue_alpha(c, b):
        pltpu.async_copy(
            hc2_v.at[b], alpha_hbm.at[pl.ds(base + c * K, K)], sem_a[b])

    def wait_alpha(c, b):
        pltpu.make_async_copy(
            hc2_v.at[b], alpha_hbm.at[pl.ds(base + c * K, K)], sem_a[b]).wait()

    def compute_alpha(b):
        for j in range(K // 16):
            sl = pl.ds(j * 16, 16)
            hs = plsc.load_gather(hsum_v, [srcc2_v[b, sl]])
            hc2_v[b, sl] = hc2_v[b, sl] / hs

    def scale_rows(b):
        rows_v = rows[b]
        hrow = hc2_v.at[b]

        @plsc.parallel_loop(0, K, 1, unroll=4)
        def _(r):
            a = plsc.load_gather(hrow, [jnp.full((16,), 0, jnp.int32) + r])
            for jj in range(N_HIDDEN // 16):
                csl = pl.ds(jj * 16, 16)
                rows_v[r, csl] = rows_v[r, csl] * a

    def step(c, b):
        @pl.when(c >= 1)
        def _():
            wait_scatter(c - 1, 1 - b)
            wait_alpha(c - 1, 1 - b)

        @pl.when(c + 1 < CHUNKS)
        def _():
            issue_stage(c + 1, 1 - b)

        wait_gather(c, b)
        compute_alpha(b)
        issue_alpha(c, b)

        @pl.when(c + 1 < CHUNKS)
        def _():
            wait_stage(c + 1, 1 - b)
            issue_gather(c + 1, 1 - b)

        scale_rows(b)
        issue_scatter(c, b)

    issue_stage(0, 0)
    wait_stage(0, 0)
    issue_gather(0, 0)

    def pair(c0, carry):
        for b in range(2):
            step(2 * c0 + b, b)
        return carry

    lax.fori_loop(0, PAIRS, pair, 0)
    step(CHUNKS - 1, 0)
    wait_scatter(CHUNKS - 1, 0)
    wait_alpha(CHUNKS - 1, 0)

    plsc.subcore_barrier()
    pltpu.sync_copy(out_sh.at[rsl], out_parts_hbm.at[cid, rsl])

    @pl.when(sid == 0)
    def _():
        pltpu.sync_copy(out_sh.at[tsl], out_parts_hbm.at[cid, tsl])


def _sc2(h, hsum_parts, src, dst, z, zeros2):
    return pl.kernel(
        _sc2_body,
        out_type=[
            jax.ShapeDtypeStruct((N_EDGES,), jnp.float32),
            jax.ShapeDtypeStruct((NC, N_NODES, N_HIDDEN), jnp.float32),
        ],
        mesh=_SC_MESH,
        compiler_params=pltpu.CompilerParams(needs_layout_passes=False),
        scratch_types=[
            pltpu.VMEM((N_NODES,), jnp.float32),
            pltpu.VMEM((N_NODES,), jnp.float32),
            pltpu.VMEM((2, K), jnp.int32),
            pltpu.VMEM((2, K), jnp.int32),
            pltpu.VMEM((2, K), jnp.float32),
            pltpu.VMEM((K, N_HIDDEN), jnp.float32),
            pltpu.VMEM((K, N_HIDDEN), jnp.float32),
            pltpu.SemaphoreType.DMA,
            pltpu.SemaphoreType.DMA,
            pltpu.SemaphoreType.DMA,
            pltpu.SemaphoreType.DMA,
            pltpu.SemaphoreType.DMA,
            pltpu.SemaphoreType.DMA,
            pltpu.SemaphoreType.DMA,
            pltpu.SemaphoreType.DMA,
            pltpu.VMEM_SHARED((N_NODES, N_HIDDEN), jnp.float32),
        ],
    )(h, hsum_parts, src, dst, z, zeros2)


@jax.jit
def _run(x, edge_index, W, a_w, a_b):
    src = edge_index[0, :].astype(jnp.int32)
    dst = edge_index[1, :].astype(jnp.int32)

    a1 = a_w[0, :N_HIDDEN]
    a2 = a_w[0, N_HIDDEN:]
    Apad = jnp.zeros((N_HIDDEN, N_HIDDEN), jnp.float32)
    Apad = Apad.at[:, 0].set(a1).at[:, 1].set(a2)
    bvec = jnp.zeros((1, N_HIDDEN), jnp.float32).at[0, 0].set(a_b[0])

    z, S = _tc1(x, W.T, Apad, bvec)
    s1 = S[:, 0]
    s2 = S[:, 1]

    zeros1 = jnp.zeros((N_NODES,), jnp.float32)
    h, hsum_parts = _sc1(s1, s2, src, dst, zeros1)

    zeros2 = jnp.zeros((N_NODES, N_HIDDEN), jnp.float32)
    alpha, out_parts = _sc2(h, hsum_parts, src, dst, z, zeros2)

    out = _tc2(out_parts[0], out_parts[1])
    return out, alpha


def kernel(x, edge_index, W, a_w, a_b):
    return _run(x, edge_index, W, a_w, a_b)
